# Initial kernel scaffold; baseline (speedup 1.0000x reference)
import jax
import jax.numpy as jnp
from jax.experimental import pallas as pl


def kernel(input, weight):
    raise NotImplementedError("write your pallas kernel here")


# speedup vs baseline: 4.9047x; 4.9047x over previous
import functools

import jax
import jax.numpy as jnp
from jax import lax
from jax.experimental import pallas as pl
from jax.experimental.pallas import tpu as pltpu
from jax.experimental.pallas import tpu_sc as plsc


def _gather_fn(n, d, chunk):
    info = plsc.get_sparse_core_info()
    nc, ns = info.num_cores, info.num_subcores
    nw = nc * ns
    per_w = n // nw
    nchunks = per_w // chunk
    assert per_w % chunk == 0 and n % nw == 0

    mesh = plsc.VectorSubcoreMesh(core_axis_name="c", subcore_axis_name="s")

    @functools.partial(
        pl.kernel,
        out_type=jax.ShapeDtypeStruct((n, d), jnp.float32),
        mesh=mesh,
        scratch_types=[
            pltpu.VMEM((chunk,), jnp.int32),
            pltpu.VMEM((chunk, d), jnp.float32),
            pltpu.SemaphoreType.DMA,
        ],
        compiler_params=pltpu.CompilerParams(use_tc_tiling_on_sc=False),
    )
    def run(idx_hbm, table_hbm, out_hbm, idx_v, rows_v, sem):
        wid = lax.axis_index("s") * nc + lax.axis_index("c")
        base = wid * per_w

        def body(i, carry):
            off = pl.multiple_of(base + i * chunk, 8)
            pltpu.sync_copy(idx_hbm.at[pl.ds(off, chunk)], idx_v)
            pltpu.async_copy(table_hbm.at[idx_v], rows_v, sem).wait()
            pltpu.sync_copy(rows_v, out_hbm.at[pl.ds(off, chunk)])
            return carry

        lax.fori_loop(0, nchunks, body, 0)

    return run


def kernel(input, weight):
    b, h = input.shape
    v, d = weight.shape
    n = b * h
    flat_idx = input.reshape(n).astype(jnp.int32)
    out = _gather_fn(n, d, 1600)(flat_idx, weight)
    return out.reshape(b, h, d)


# speedup vs baseline: 5.0505x; 1.0297x over previous
import functools

import jax
import jax.numpy as jnp
from jax import lax
from jax.experimental import pallas as pl
from jax.experimental.pallas import tpu as pltpu
from jax.experimental.pallas import tpu_sc as plsc

_NBUF = 2


def _gather_fn(n, d, chunk):
    info = plsc.get_sparse_core_info()
    nc, ns = info.num_cores, info.num_subcores
    nw = nc * ns
    per_w = n // nw
    nchunks = per_w // chunk
    nsteps = nchunks // _NBUF
    assert per_w % chunk == 0 and n % nw == 0 and nchunks % _NBUF == 0
    assert nsteps >= 3

    mesh = plsc.VectorSubcoreMesh(core_axis_name="c", subcore_axis_name="s")

    @functools.partial(
        pl.kernel,
        out_type=jax.ShapeDtypeStruct((n, d), jnp.float32),
        mesh=mesh,
        scratch_types=[
            pltpu.VMEM((_NBUF, chunk), jnp.int32),
            pltpu.VMEM((_NBUF, chunk, d), jnp.float32),
            [pltpu.SemaphoreType.DMA] * _NBUF,
            [pltpu.SemaphoreType.DMA] * _NBUF,
            [pltpu.SemaphoreType.DMA] * _NBUF,
        ],
        compiler_params=pltpu.CompilerParams(use_tc_tiling_on_sc=False),
    )
    def run(idx_hbm, table_hbm, out_hbm, idx_v, rows_v, isems, gsems, osems):
        wid = lax.axis_index("s") * nc + lax.axis_index("c")
        base = wid * per_w

        def off(g):
            return pl.multiple_of(base + g * chunk, 8)

        def start_idx(b, g):
            pltpu.async_copy(idx_hbm.at[pl.ds(off(g), chunk)], idx_v.at[b],
                             isems[b])

        def wait_idx(b):
            pltpu.make_async_copy(idx_hbm.at[pl.ds(off(0), chunk)],
                                  idx_v.at[b], isems[b]).wait()

        def start_gather(b):
            pltpu.async_copy(table_hbm.at[idx_v.at[b]], rows_v.at[b], gsems[b])

        def wait_gather(b):
            pltpu.make_async_copy(table_hbm.at[idx_v.at[b]], rows_v.at[b],
                                  gsems[b]).wait()

        def start_out(b, g):
            pltpu.async_copy(rows_v.at[b], out_hbm.at[pl.ds(off(g), chunk)],
                             osems[b])

        def wait_out(b):
            pltpu.make_async_copy(rows_v.at[b],
                                  out_hbm.at[pl.ds(off(0), chunk)],
                                  osems[b]).wait()


        for b in range(_NBUF):
            start_idx(b, b)
        for g in range(_NBUF):
            b = g
            wait_idx(b)
            start_gather(b)
            if g > 0:
                wait_gather(g - 1)
                start_out(g - 1, g - 1)
                start_idx(g - 1, g - 1 + _NBUF)

        def step_body(s, c):
            for b in range(_NBUF):
                g = s * _NBUF + b
                wait_idx(b)
                wait_out(b)
                start_gather(b)
                pb = (b - 1) % _NBUF
                wait_gather(pb)
                start_out(pb, g - 1)
                start_idx(pb, g - 1 + _NBUF)
            return c

        lax.fori_loop(1, nsteps - 1, step_body, 0)

        for b in range(_NBUF):
            g = (nsteps - 1) * _NBUF + b
            wait_idx(b)
            wait_out(b)
            start_gather(b)
            pb = (b - 1) % _NBUF
            wait_gather(pb)
            start_out(pb, g - 1)
            if g - 1 + _NBUF < nchunks:
                start_idx(pb, g - 1 + _NBUF)
        bl = (nchunks - 1) % _NBUF
        wait_gather(bl)
        start_out(bl, nchunks - 1)
        for b in range(_NBUF):
            wait_out(b)

    return run


def kernel(input, weight):
    b, h = input.shape
    v, d = weight.shape
    n = b * h
    flat_idx = input.reshape(n).astype(jnp.int32)
    out = _gather_fn(n, d, 1600)(flat_idx, weight)
    return out.reshape(b, h, d)
